# baseline (speedup 1.0000x reference)
import functools

import jax
import jax.numpy as jnp
from jax import lax
from jax.experimental import pallas as pl
from jax.experimental.pallas import tpu as pltpu
from jax.experimental.pallas import tpu_sc as plsc

NI = 10000
NT = 10000
E = 320000
F = 128
NC = 2
NS = 16
L = 16

E_PAD = 327680
ER = E_PAD // 128
NWR = 10240
DSTRIPE = NWR // NS

CR1 = 16
A_CHUNKS = ER // (NC * NS) // CR1

C2 = 128
ER2 = E_PAD // C2
B_N = ER2 // (NC * NS)
NSLOT = 4


def _prep_body(xi, xt, wi, bi, ls, wt, bt, ld, h_ref, asrc_ref, adst_ref, m_ref):
    xiv = xi[...]
    h = lax.dot_general(xiv, wi[...], (((1,), (1,)), ((), ())),
                        preferred_element_type=jnp.float32) + bi[...]
    h_ref[...] = h
    a_s = jnp.sum(h * ls[...], axis=1)
    asrc_ref[...] = jnp.concatenate([a_s, jnp.zeros((NWR - NI,), jnp.float32)])
    u = jnp.dot(ld[...], wt[...], preferred_element_type=jnp.float32)
    c = jnp.sum(bt[...] * ld[...])
    a_d = jnp.sum(xt[...] * u, axis=1) + c
    adst_ref[...] = jnp.concatenate([a_d, jnp.zeros((NWR - NT,), jnp.float32)])
    m = jnp.maximum(jnp.max(a_s) + jnp.max(a_d), 0.0)
    m_ref[...] = jnp.full((L,), m, jnp.float32)


def _post_body(p_ref, xt_ref, o_ref):
    o_ref[...] = jnp.maximum(p_ref[0, :NT] + p_ref[1, :NT], 0.0) + xt_ref[...]


def _sc_den_body(srcf, dst2d, asrc_hbm, adst_hbm, m_hbm, denp_hbm, ex_hbm,
                 s1a, s1b, s1c, s1d, s1e,
                 r1a, r1b, r1c, r1d, r1e, x1a, x1b, x1c, x1d, x1e,
                 asrc_t, adst_t, m_t, zbuf, den_sh,
                 sem_i, sem_i2, sem_s):
    src1 = [s1a, s1b, s1c, s1d, s1e]
    dst1r = [r1a, r1b, r1c, r1d, r1e]
    ex1 = [x1a, x1b, x1c, x1d, x1e]
    c = lax.axis_index("c")
    s = lax.axis_index("s")
    w = c * NS + s

    zero16 = jnp.zeros((L,), jnp.float32)

    def _z(i, _):
        zbuf[pl.ds(i * L, L)] = zero16
        return 0
    lax.fori_loop(0, DSTRIPE // L, _z, 0)
    pltpu.sync_copy(zbuf, den_sh.at[pl.ds(s * DSTRIPE, DSTRIPE)])

    pltpu.sync_copy(asrc_hbm, asrc_t)
    pltpu.sync_copy(adst_hbm, adst_t)
    pltpu.sync_copy(m_hbm, m_t)
    plsc.subcore_barrier()
    mvec = m_t[...]

    def _fire_a_idx(k):
        r0 = (w * A_CHUNKS + k) * CR1
        d1 = pltpu.async_copy(srcf.at[pl.ds(r0 * 128, CR1 * 128)], src1[k],
                              sem_i if k % 2 == 0 else sem_i2)
        d3 = pltpu.async_copy(dst2d.at[pl.ds(r0, CR1)], dst1r[k],
                              sem_i if k % 2 == 0 else sem_i2)
        return (d1, d3)

    idescs = [_fire_a_idx(0), _fire_a_idx(1)]
    for k in range(A_CHUNKS):
        r0 = (w * A_CHUNKS + k) * CR1
        for d in idescs[k]:
            d.wait()
        if k + 2 < A_CHUNKS:
            idescs.append(_fire_a_idx(k + 2))

        def _row(jj, _, k=k):
            for g in range(128 // L):
                off = g * L
                d16 = dst1r[k][jj, pl.ds(off, L)]
                s16 = src1[k][pl.ds(jj * 128 + off, L)]
                a16 = plsc.load_gather(asrc_t, [s16]) + plsc.load_gather(adst_t, [d16])
                a16 = jnp.maximum(a16, 0.2 * a16)
                ex1[k][pl.ds(jj * 128 + off, L)] = jnp.exp(a16 - mvec)
            return 0
        lax.fori_loop(0, CR1, _row, 0)

        for j0 in range(0, CR1, 8):
            dl = [pltpu.async_copy(ex1[k].at[pl.ds(j * 128, 128)],
                                   den_sh.at[dst1r[k].at[j]], sem_s, add=True)
                  for j in range(j0, j0 + 8)]
            for d in dl:
                d.wait()
        pltpu.async_copy(ex1[k], ex_hbm.at[pl.ds(r0 * 128, CR1 * 128)], sem_s).wait()

    plsc.subcore_barrier()
    pltpu.sync_copy(den_sh.at[pl.ds(s * DSTRIPE, DSTRIPE)],
                    denp_hbm.at[pl.ds(c * NWR + s * DSTRIPE, DSTRIPE)])


def _sc_msg_body(src64, dst64, ex64, denp_hbm, h_hbm, out_hbm,
                 den_t, da, db, src2, dst2, ex2, w2,
                 rows0, rows1,
                 den_sh, out_sh,
                 si0, si1, si2, si3, sg0, sg1, ss0, ss1):
    c = lax.axis_index("c")
    s = lax.axis_index("s")
    w = c * NS + s
    rows = [rows0, rows1]
    sem_i = [si0, si1, si2, si3]
    sem_g = [sg0, sg1]
    sem_s = [ss0, ss1]

    zero16 = jnp.zeros((L,), jnp.float32)

    pltpu.sync_copy(denp_hbm.at[pl.ds(s * DSTRIPE, DSTRIPE)], da)
    pltpu.sync_copy(denp_hbm.at[pl.ds(NWR + s * DSTRIPE, DSTRIPE)], db)

    def _m(g, _):
        off = g * L
        da[pl.ds(off, L)] = da[pl.ds(off, L)] + db[pl.ds(off, L)]
        return 0
    lax.fori_loop(0, DSTRIPE // L, _m, 0)
    pltpu.sync_copy(da, den_sh.at[pl.ds(s * DSTRIPE, DSTRIPE)])

    def _zrow(e, _):
        for j in range(F // L):
            rows0[e, pl.ds(j * L, L)] = zero16
        return 0
    lax.fori_loop(0, C2, _zrow, 0)
    for i in range(DSTRIPE // C2):
        pltpu.sync_copy(rows0, out_sh.at[pl.ds(s * DSTRIPE + i * C2, C2)])

    plsc.subcore_barrier()
    pltpu.sync_copy(den_sh, den_t)

    def _fire_idx(m, q):
        erow = w * B_N + m
        pltpu.async_copy(src64.at[pl.ds(erow, 1)], src2.at[pl.ds(q, 1)], sem_i[q])
        pltpu.async_copy(dst64.at[pl.ds(erow, 1)], dst2.at[pl.ds(q, 1)], sem_i[q])
        pltpu.async_copy(ex64.at[pl.ds(erow, 1)], ex2.at[pl.ds(q, 1)], sem_i[q])

    def _drain_idx(q):
        pltpu.make_async_copy(src64.at[pl.ds(0, 1)], src2.at[pl.ds(q, 1)], sem_i[q]).wait()
        pltpu.make_async_copy(dst64.at[pl.ds(0, 1)], dst2.at[pl.ds(q, 1)], sem_i[q]).wait()
        pltpu.make_async_copy(ex64.at[pl.ds(0, 1)], ex2.at[pl.ds(q, 1)], sem_i[q]).wait()

    def _drain_rows(rb, sems):
        pltpu.make_async_copy(h_hbm.at[pl.ds(0, C2)], rows[rb], sems[rb]).wait()

    def _compute_w(q):
        for g in range(C2 // L):
            off = g * L
            d16 = dst2[q, pl.ds(off, L)]
            den16 = plsc.load_gather(den_t, [d16])
            w2[q, pl.ds(off, L)] = ex2[q, pl.ds(off, L)] / (den16 + 1e-16)

    def _fire_gather(q, rb):
        pltpu.async_copy(h_hbm.at[src2.at[q]], rows[rb], sem_g[rb])

    def _scale(q, rb):
        def _se(e, _, q=q, rb=rb):
            wsplat = plsc.load_gather(w2, [jnp.full((L,), q, jnp.int32),
                                           jnp.full((L,), e, jnp.int32)])
            r = rows[rb]
            for j in range(F // L):
                r[e, pl.ds(j * L, L)] = r[e, pl.ds(j * L, L)] * wsplat
            return 0
        lax.fori_loop(0, C2, _se, 0)

    def _fire_scatter(q, rb):
        pltpu.async_copy(rows[rb], out_sh.at[dst2.at[q]], sem_s[rb], add=True)

    _fire_idx(0, 0)
    _fire_idx(1, 1)
    _drain_idx(0)
    _compute_w(0)
    _fire_gather(0, 0)

    nlast = B_N // NSLOT - 1

    def _body(i2, _):
        for k in range(NSLOT):
            rb = k % 2
            q1 = (k + 1) % NSLOT
            q2 = (k + 2) % NSLOT
            if k >= 1:
                _drain_rows(1 - rb, sem_s)
            else:
                @pl.when(i2 > 0)
                def _():
                    _drain_rows(1 - rb, sem_s)
            if k < 2:
                _fire_idx(i2 * NSLOT + k + 2, q2)
            else:
                @pl.when(i2 < nlast)
                def _():
                    _fire_idx(i2 * NSLOT + k + 2, q2)
            if k < 3:
                _drain_idx(q1)
                _compute_w(q1)
                _fire_gather(q1, 1 - rb)
            else:
                @pl.when(i2 < nlast)
                def _():
                    _drain_idx(q1)
                    _compute_w(q1)
                    _fire_gather(q1, 1 - rb)
            _drain_rows(rb, sem_g)
            _scale(k, rb)
            _fire_scatter(k, rb)
        return 0
    lax.fori_loop(0, B_N // NSLOT, _body, 0)

    _drain_rows((B_N - 1) % 2, sem_s)

    plsc.subcore_barrier()
    pltpu.sync_copy(out_sh.at[pl.ds(s * DSTRIPE, DSTRIPE)],
                    out_hbm.at[c, pl.ds(s * DSTRIPE, DSTRIPE)])


def kernel(x_ingredient, x_taste, edge_index, proj_ing_w, proj_ing_b,
           proj_taste_w, proj_taste_b, lin_src, lin_dst, k_lin_w, k_lin_b, q):
    ls = lin_src.reshape(1, F)
    ld = lin_dst.reshape(1, F)

    h, asrc, adst, m = pl.pallas_call(
        _prep_body,
        out_shape=[
            jax.ShapeDtypeStruct((NI, F), jnp.float32),
            jax.ShapeDtypeStruct((NWR,), jnp.float32),
            jax.ShapeDtypeStruct((NWR,), jnp.float32),
            jax.ShapeDtypeStruct((L,), jnp.float32),
        ],
    )(x_ingredient, x_taste, proj_ing_w, proj_ing_b, ls, proj_taste_w,
      proj_taste_b, ld)

    src = edge_index[0]
    dst = edge_index[1]
    pids = jnp.arange(E_PAD - E, dtype=jnp.int32)
    srcp = jnp.concatenate([src, pids % 997])
    dstp = jnp.concatenate([dst, NT + L + (pids % 128)])
    dst2d = dstp.reshape(ER, 128)

    mesh = plsc.VectorSubcoreMesh(core_axis_name="c", subcore_axis_name="s",
                                  num_cores=NC, num_subcores=NS)

    den_launch = functools.partial(
        pl.kernel,
        out_type=[
            jax.ShapeDtypeStruct((NC * NWR,), jnp.float32),
            jax.ShapeDtypeStruct((E_PAD,), jnp.float32),
        ],
        mesh=mesh,
        compiler_params=pltpu.CompilerParams(needs_layout_passes=False),
        scratch_types=(
            [pltpu.VMEM((CR1 * 128,), jnp.int32)] * A_CHUNKS
            + [pltpu.VMEM((CR1, 128), jnp.int32)] * A_CHUNKS
            + [pltpu.VMEM((CR1 * 128,), jnp.float32)] * A_CHUNKS
            + [
                pltpu.VMEM((NWR,), jnp.float32),
                pltpu.VMEM((NWR,), jnp.float32),
                pltpu.VMEM((L,), jnp.float32),
                pltpu.VMEM((DSTRIPE,), jnp.float32),
                pltpu.VMEM_SHARED((NWR,), jnp.float32),
                pltpu.SemaphoreType.DMA,
                pltpu.SemaphoreType.DMA,
                pltpu.SemaphoreType.DMA,
            ]
        ),
    )(_sc_den_body)
    denp, exbuf = den_launch(srcp, dst2d, asrc, adst, m)

    src64 = srcp.reshape(ER2, C2)
    dst64 = dstp.reshape(ER2, C2)
    ex64 = exbuf.reshape(ER2, C2)

    msg_launch = functools.partial(
        pl.kernel,
        out_type=jax.ShapeDtypeStruct((NC, NWR, F), jnp.float32),
        mesh=mesh,
        compiler_params=pltpu.CompilerParams(needs_layout_passes=False),
        scratch_types=[
            pltpu.VMEM((NWR,), jnp.float32),
            pltpu.VMEM((DSTRIPE,), jnp.float32),
            pltpu.VMEM((DSTRIPE,), jnp.float32),
            pltpu.VMEM((NSLOT, C2), jnp.int32),
            pltpu.VMEM((NSLOT, C2), jnp.int32),
            pltpu.VMEM((NSLOT, C2), jnp.float32),
            pltpu.VMEM((NSLOT, C2), jnp.float32),
            pltpu.VMEM((C2, F), jnp.float32),
            pltpu.VMEM((C2, F), jnp.float32),
            pltpu.VMEM_SHARED((NWR,), jnp.float32),
            pltpu.VMEM_SHARED((NWR, F), jnp.float32),
        ] + [pltpu.SemaphoreType.DMA] * 8,
    )(_sc_msg_body)
    partials = msg_launch(src64, dst64, ex64, denp, h)

    out_taste = pl.pallas_call(
        _post_body,
        out_shape=jax.ShapeDtypeStruct((NT, F), jnp.float32),
    )(partials, x_taste)

    return (x_ingredient, out_taste)


# speedup vs baseline: 1.1104x; 1.1104x over previous
import functools

import jax
import jax.numpy as jnp
from jax import lax
from jax.experimental import pallas as pl
from jax.experimental.pallas import tpu as pltpu
from jax.experimental.pallas import tpu_sc as plsc

NI = 10000
NT = 10000
E = 320000
F = 128
NC = 2
NS = 16
L = 16

E_PAD = 327680
ER = E_PAD // 128
NWR = 10240
DSTRIPE = NWR // NS

CR1 = 16
A_CHUNKS = ER // (NC * NS) // CR1

C2 = 64
ER2 = E_PAD // C2
B_N = ER2 // (NC * NS)
NSLOT = 4


def _prep_body(xi, xt, wi, bi, ls, wt, bt, ld, h_ref, asrc_ref, adst_ref, m_ref):
    xiv = xi[...]
    h = lax.dot_general(xiv, wi[...], (((1,), (1,)), ((), ())),
                        preferred_element_type=jnp.float32) + bi[...]
    h_ref[...] = h
    a_s = jnp.sum(h * ls[...], axis=1)
    asrc_ref[...] = jnp.concatenate([a_s, jnp.zeros((NWR - NI,), jnp.float32)])
    u = jnp.dot(ld[...], wt[...], preferred_element_type=jnp.float32)
    c = jnp.sum(bt[...] * ld[...])
    a_d = jnp.sum(xt[...] * u, axis=1) + c
    adst_ref[...] = jnp.concatenate([a_d, jnp.zeros((NWR - NT,), jnp.float32)])
    m = jnp.maximum(jnp.max(a_s) + jnp.max(a_d), 0.0)
    m_ref[...] = jnp.full((L,), m, jnp.float32)


def _post_body(p_ref, den_ref, xt_ref, o_ref):
    o_ref[...] = jnp.maximum((p_ref[0, :NT] + p_ref[1, :NT]) / den_ref[...],
                             0.0) + xt_ref[...]


def _sc_den_body(srcf, dst2d, asrc_hbm, adst_hbm, m_hbm, denp_hbm, ex_hbm,
                 s1a, s1b, s1c, s1d, s1e,
                 r1a, r1b, r1c, r1d, r1e, x1a, x1b, x1c, x1d, x1e,
                 asrc_t, adst_t, m_t, zbuf, den_sh,
                 sem_i, sem_i2, sem_s):
    src1 = [s1a, s1b, s1c, s1d, s1e]
    dst1r = [r1a, r1b, r1c, r1d, r1e]
    ex1 = [x1a, x1b, x1c, x1d, x1e]
    c = lax.axis_index("c")
    s = lax.axis_index("s")
    w = c * NS + s

    zero16 = jnp.zeros((L,), jnp.float32)

    def _z(i, _):
        zbuf[pl.ds(i * L, L)] = zero16
        return 0
    lax.fori_loop(0, DSTRIPE // L, _z, 0)
    pltpu.sync_copy(zbuf, den_sh.at[pl.ds(s * DSTRIPE, DSTRIPE)])

    pltpu.sync_copy(asrc_hbm, asrc_t)
    pltpu.sync_copy(adst_hbm, adst_t)
    pltpu.sync_copy(m_hbm, m_t)
    plsc.subcore_barrier()
    mvec = m_t[...]

    def _fire_a_idx(k):
        r0 = (w * A_CHUNKS + k) * CR1
        d1 = pltpu.async_copy(srcf.at[pl.ds(r0 * 128, CR1 * 128)], src1[k],
                              sem_i if k % 2 == 0 else sem_i2)
        d3 = pltpu.async_copy(dst2d.at[pl.ds(r0, CR1)], dst1r[k],
                              sem_i if k % 2 == 0 else sem_i2)
        return (d1, d3)

    idescs = [_fire_a_idx(0), _fire_a_idx(1)]
    for k in range(A_CHUNKS):
        r0 = (w * A_CHUNKS + k) * CR1
        for d in idescs[k]:
            d.wait()
        if k + 2 < A_CHUNKS:
            idescs.append(_fire_a_idx(k + 2))

        def _row(jj, _, k=k):
            for g in range(128 // L):
                off = g * L
                d16 = dst1r[k][jj, pl.ds(off, L)]
                s16 = src1[k][pl.ds(jj * 128 + off, L)]
                a16 = plsc.load_gather(asrc_t, [s16]) + plsc.load_gather(adst_t, [d16])
                a16 = jnp.maximum(a16, 0.2 * a16)
                ex1[k][pl.ds(jj * 128 + off, L)] = jnp.exp(a16 - mvec)
            return 0
        lax.fori_loop(0, CR1, _row, 0)

        for j0 in range(0, CR1, 8):
            dl = [pltpu.async_copy(ex1[k].at[pl.ds(j * 128, 128)],
                                   den_sh.at[dst1r[k].at[j]], sem_s, add=True)
                  for j in range(j0, j0 + 8)]
            for d in dl:
                d.wait()
        pltpu.async_copy(ex1[k], ex_hbm.at[pl.ds(r0 * 128, CR1 * 128)], sem_s).wait()

    plsc.subcore_barrier()
    pltpu.sync_copy(den_sh.at[pl.ds(s * DSTRIPE, DSTRIPE)],
                    denp_hbm.at[pl.ds(c * NWR + s * DSTRIPE, DSTRIPE)])


def _sc_msg_body(src64, dst64, ex64, h_hbm, out_hbm,
                 src2, dst2, ex2,
                 rows0, rows1, rows2, rows3,
                 out_sh,
                 si0, si1, si2, si3, sg0, sg1, sg2, sg3, ss0, ss1, ss2, ss3):
    c = lax.axis_index("c")
    s = lax.axis_index("s")
    w = c * NS + s
    rows = [rows0, rows1, rows2, rows3]
    sem_i = [si0, si1, si2, si3]
    sem_g = [sg0, sg1, sg2, sg3]
    sem_s = [ss0, ss1, ss2, ss3]

    zero16 = jnp.zeros((L,), jnp.float32)

    def _zrow(e, _):
        for j in range(F // L):
            rows0[e, pl.ds(j * L, L)] = zero16
        return 0
    lax.fori_loop(0, C2, _zrow, 0)
    for i in range(DSTRIPE // C2):
        pltpu.sync_copy(rows0, out_sh.at[pl.ds(s * DSTRIPE + i * C2, C2)])

    plsc.subcore_barrier()

    def _fire_idx(m, q):
        erow = w * B_N + m
        pltpu.async_copy(src64.at[pl.ds(erow, 1)], src2.at[pl.ds(q, 1)], sem_i[q])
        pltpu.async_copy(dst64.at[pl.ds(erow, 1)], dst2.at[pl.ds(q, 1)], sem_i[q])
        pltpu.async_copy(ex64.at[pl.ds(erow, 1)], ex2.at[pl.ds(q, 1)], sem_i[q])

    def _drain_idx(q):
        pltpu.make_async_copy(src64.at[pl.ds(0, 1)], src2.at[pl.ds(q, 1)], sem_i[q]).wait()
        pltpu.make_async_copy(dst64.at[pl.ds(0, 1)], dst2.at[pl.ds(q, 1)], sem_i[q]).wait()
        pltpu.make_async_copy(ex64.at[pl.ds(0, 1)], ex2.at[pl.ds(q, 1)], sem_i[q]).wait()

    def _drain_rows(q, sems):
        pltpu.make_async_copy(h_hbm.at[pl.ds(0, C2)], rows[q], sems[q]).wait()

    def _fire_gather(q):
        pltpu.async_copy(h_hbm.at[src2.at[q]], rows[q], sem_g[q])

    def _scale(q):
        def _se(e, _, q=q):
            wsplat = plsc.load_gather(ex2, [jnp.full((L,), q, jnp.int32),
                                            jnp.full((L,), e, jnp.int32)])
            r = rows[q]
            for j in range(F // L):
                r[e, pl.ds(j * L, L)] = r[e, pl.ds(j * L, L)] * wsplat
            return 0
        lax.fori_loop(0, C2, _se, 0)

    def _fire_scatter(q):
        pltpu.async_copy(rows[q], out_sh.at[dst2.at[q]], sem_s[q], add=True)

    _fire_idx(0, 0)
    _fire_idx(1, 1)
    _drain_idx(0)
    _fire_gather(0)

    def _body(i2, _):
        for k in range(NSLOT):
            q2 = (k + 2) % NSLOT
            q1 = (k + 1) % NSLOT
            if k >= 2:
                _drain_rows(q2, sem_s)
            else:
                @pl.when(i2 > 0)
                def _():
                    _drain_rows(q2, sem_s)
            if k < 2:
                _fire_idx(i2 * NSLOT + k + 2, q2)
            else:
                @pl.when(i2 < (B_N // NSLOT) - 1)
                def _():
                    _fire_idx(i2 * NSLOT + k + 2, q2)
            if k < 3:
                _drain_idx(q1)
                _fire_gather(q1)
            else:
                @pl.when(i2 < (B_N // NSLOT) - 1)
                def _():
                    _drain_idx(q1)
                    _fire_gather(q1)
            _drain_rows(k, sem_g)
            _scale(k)
            _fire_scatter(k)
        return 0
    lax.fori_loop(0, B_N // NSLOT, _body, 0)

    _drain_rows(2, sem_s)
    _drain_rows(3, sem_s)

    plsc.subcore_barrier()
    pltpu.sync_copy(out_sh.at[pl.ds(s * DSTRIPE, DSTRIPE)],
                    out_hbm.at[c, pl.ds(s * DSTRIPE, DSTRIPE)])


def kernel(x_ingredient, x_taste, edge_index, proj_ing_w, proj_ing_b,
           proj_taste_w, proj_taste_b, lin_src, lin_dst, k_lin_w, k_lin_b, q):
    ls = lin_src.reshape(1, F)
    ld = lin_dst.reshape(1, F)

    h, asrc, adst, m = pl.pallas_call(
        _prep_body,
        out_shape=[
            jax.ShapeDtypeStruct((NI, F), jnp.float32),
            jax.ShapeDtypeStruct((NWR,), jnp.float32),
            jax.ShapeDtypeStruct((NWR,), jnp.float32),
            jax.ShapeDtypeStruct((L,), jnp.float32),
        ],
    )(x_ingredient, x_taste, proj_ing_w, proj_ing_b, ls, proj_taste_w,
      proj_taste_b, ld)

    src = edge_index[0]
    dst = edge_index[1]
    pids = jnp.arange(E_PAD - E, dtype=jnp.int32)
    srcp = jnp.concatenate([src, pids % 997])
    dstp = jnp.concatenate([dst, NT + L + (pids % 128)])
    dst2d = dstp.reshape(ER, 128)

    mesh = plsc.VectorSubcoreMesh(core_axis_name="c", subcore_axis_name="s",
                                  num_cores=NC, num_subcores=NS)

    den_launch = functools.partial(
        pl.kernel,
        out_type=[
            jax.ShapeDtypeStruct((NC * NWR,), jnp.float32),
            jax.ShapeDtypeStruct((E_PAD,), jnp.float32),
        ],
        mesh=mesh,
        compiler_params=pltpu.CompilerParams(needs_layout_passes=False),
        scratch_types=(
            [pltpu.VMEM((CR1 * 128,), jnp.int32)] * A_CHUNKS
            + [pltpu.VMEM((CR1, 128), jnp.int32)] * A_CHUNKS
            + [pltpu.VMEM((CR1 * 128,), jnp.float32)] * A_CHUNKS
            + [
                pltpu.VMEM((NWR,), jnp.float32),
                pltpu.VMEM((NWR,), jnp.float32),
                pltpu.VMEM((L,), jnp.float32),
                pltpu.VMEM((DSTRIPE,), jnp.float32),
                pltpu.VMEM_SHARED((NWR,), jnp.float32),
                pltpu.SemaphoreType.DMA,
                pltpu.SemaphoreType.DMA,
                pltpu.SemaphoreType.DMA,
            ]
        ),
    )(_sc_den_body)
    denp, exbuf = den_launch(srcp, dst2d, asrc, adst, m)

    src64 = srcp.reshape(ER2, C2)
    dst64 = dstp.reshape(ER2, C2)
    ex64 = exbuf.reshape(ER2, C2)

    msg_launch = functools.partial(
        pl.kernel,
        out_type=jax.ShapeDtypeStruct((NC, NWR, F), jnp.float32),
        mesh=mesh,
        compiler_params=pltpu.CompilerParams(needs_layout_passes=False),
        scratch_types=[
            pltpu.VMEM((NSLOT, C2), jnp.int32),
            pltpu.VMEM((NSLOT, C2), jnp.int32),
            pltpu.VMEM((NSLOT, C2), jnp.float32),
            pltpu.VMEM((C2, F), jnp.float32),
            pltpu.VMEM((C2, F), jnp.float32),
            pltpu.VMEM((C2, F), jnp.float32),
            pltpu.VMEM((C2, F), jnp.float32),
            pltpu.VMEM_SHARED((NWR, F), jnp.float32),
        ] + [pltpu.SemaphoreType.DMA] * 12,
    )(_sc_msg_body)
    partials = msg_launch(src64, dst64, ex64, h)

    den2 = (denp[:NWR] + denp[NWR:])[:NT, None] + 1e-16
    out_taste = pl.pallas_call(
        _post_body,
        out_shape=jax.ShapeDtypeStruct((NT, F), jnp.float32),
    )(partials, den2, x_taste)

    return (x_ingredient, out_taste)
